# Initial kernel scaffold; baseline (speedup 1.0000x reference)
import jax
import jax.numpy as jnp
from jax.experimental import pallas as pl


def kernel(x_real, x_imag, magnitude_kernel, phase_kernel):
    raise NotImplementedError("write your pallas kernel here")


# speedup vs baseline: 2.0734x; 2.0734x over previous
import functools

import numpy as np
import jax
import jax.numpy as jnp
from jax.experimental import pallas as pl
from jax.experimental.pallas import tpu as pltpu

_N = 4096


def _dft_mats():
    i = np.arange(_N)
    prod = (i[:, None].astype(np.int64) * i[None, :]) % _N
    theta = prod.astype(np.float64) * (2.0 * np.pi / _N)
    sign = np.where(i % 2 == 0, 1.0, -1.0)[:, None]
    c = sign * np.cos(theta) / _N
    s = sign * np.sin(theta) / _N
    bf = jnp.bfloat16
    return (c.astype(bf), s.astype(bf),
            np.ascontiguousarray(c.T).astype(bf),
            np.ascontiguousarray(s.T).astype(bf))


_C, _S, _CT, _ST = _dft_mats()


def _pointwise_body(xr_ref, xi_ref, mk_ref, pk_ref, or_ref, oi_ref):
    xr = xr_ref[...]
    xi = xi_ref[...]
    mag = jnp.sqrt(xr * xr + xi * xi) * mk_ref[...]
    ph = jnp.arctan2(xi, xr) * pk_ref[...]
    or_ref[...] = (mag * jnp.cos(ph)).astype(jnp.bfloat16)
    oi_ref[...] = (mag * jnp.sin(ph)).astype(jnp.bfloat16)


def _stage1_body(c_ref, s_ref, xr_ref, xi_ref, p_ref, q_ref):
    c = c_ref[...]
    s = s_ref[...]
    xr = xr_ref[...]
    xi = xi_ref[...]
    p_ref[...] = (jnp.dot(c, xr, preferred_element_type=jnp.float32)
                  - jnp.dot(s, xi, preferred_element_type=jnp.float32)
                  ).astype(jnp.bfloat16)
    q_ref[...] = (jnp.dot(c, xi, preferred_element_type=jnp.float32)
                  + jnp.dot(s, xr, preferred_element_type=jnp.float32)
                  ).astype(jnp.bfloat16)


def _stage2_body(p_ref, q_ref, ct_ref, st_ref, o_ref):
    o_ref[...] = (jnp.dot(p_ref[...], ct_ref[...], preferred_element_type=jnp.float32)
                  - jnp.dot(q_ref[...], st_ref[...], preferred_element_type=jnp.float32))


_BM = 512
_BN = 512
_PW_ROWS = 256


@functools.partial(jax.jit, static_argnums=())
def kernel(x_real, x_imag, magnitude_kernel, phase_kernel):
    xr = x_real.reshape(_N, _N)
    xi = x_imag.reshape(_N, _N)
    mk = magnitude_kernel.reshape(_N, _N)
    pk = phase_kernel.reshape(_N, _N)

    pw_spec = pl.BlockSpec((_PW_ROWS, _N), lambda i: (i, 0))
    Xr, Xi = pl.pallas_call(
        _pointwise_body,
        grid=(_N // _PW_ROWS,),
        in_specs=[pw_spec] * 4,
        out_specs=[pw_spec] * 2,
        out_shape=[jax.ShapeDtypeStruct((_N, _N), jnp.bfloat16)] * 2,
        compiler_params=pltpu.CompilerParams(
            dimension_semantics=("parallel",),
            vmem_limit_bytes=60 * 1024 * 1024,
        ),
    )(xr, xi, mk, pk)

    lhs_spec = pl.BlockSpec((_BM, _N), lambda i, j: (i, 0))
    rhs_spec = pl.BlockSpec((_N, _BN), lambda i, j: (0, j))
    out_spec = pl.BlockSpec((_BM, _BN), lambda i, j: (i, j))

    P, Q = pl.pallas_call(
        _stage1_body,
        grid=(_N // _BM, _N // _BN),
        in_specs=[lhs_spec, lhs_spec, rhs_spec, rhs_spec],
        out_specs=[out_spec, out_spec],
        out_shape=[jax.ShapeDtypeStruct((_N, _N), jnp.bfloat16)] * 2,
        compiler_params=pltpu.CompilerParams(
            dimension_semantics=("parallel", "arbitrary"),
            vmem_limit_bytes=60 * 1024 * 1024,
        ),
    )(_C, _S, Xr, Xi)

    out = pl.pallas_call(
        _stage2_body,
        grid=(_N // _BM, _N // _BN),
        in_specs=[lhs_spec, lhs_spec, rhs_spec, rhs_spec],
        out_specs=out_spec,
        out_shape=jax.ShapeDtypeStruct((_N, _N), jnp.float32),
        compiler_params=pltpu.CompilerParams(
            dimension_semantics=("parallel", "arbitrary"),
            vmem_limit_bytes=60 * 1024 * 1024,
        ),
    )(P, Q, _CT, _ST)

    return out.reshape(1, _N, _N)
